# Initial kernel scaffold; baseline (speedup 1.0000x reference)
import jax
import jax.numpy as jnp
from jax.experimental import pallas as pl


def kernel(x, edge_index, params):
    raise NotImplementedError("write your pallas kernel here")


# speedup vs baseline: 3.3517x; 3.3517x over previous
import functools

import jax
import jax.numpy as jnp
from jax import lax
from jax.experimental import pallas as pl
from jax.experimental.pallas import tpu as pltpu
from jax.experimental.pallas import tpu_sc as plsc

SUB = 125
NW = 32
EPS = 1e-5
_SC_PARAMS = pltpu.CompilerParams(use_tc_tiling_on_sc=False)
_MESH = dict(core_axis_name="c", subcore_axis_name="s")


def _nsub(rpw):
    return 4 if rpw % 4 == 0 else (2 if rpw % 2 == 0 else 1)


def _sc_gather_concat(x, dst2, src2):
    n, w = x.shape
    assert w % 16 == 0
    rows, sub = dst2.shape
    e = rows * sub
    rpw = rows // NW
    ns = _nsub(rpw)
    ch = ns * sub
    mesh = plsc.VectorSubcoreMesh(**_MESH)

    @functools.partial(
        pl.kernel, mesh=mesh,
        out_type=jax.ShapeDtypeStruct((e, 2 * w), jnp.float32),
        compiler_params=_SC_PARAMS,
        scratch_types=[
            pltpu.VMEM((ns, sub), jnp.int32),
            pltpu.VMEM((ns, sub), jnp.int32),
            pltpu.VMEM((ch, w), jnp.float32),
            pltpu.VMEM((ch, w), jnp.float32),
            pltpu.SemaphoreType.DMA,
        ],
    )
    def k(x_hbm, dst_hbm, src_hbm, out_hbm, idxd, idxs, bi, bj, sem):
        wid = lax.axis_index("s") * 2 + lax.axis_index("c")
        row0 = wid * rpw

        def chunk(c, _):
            r0 = row0 + c * ns
            pltpu.sync_copy(dst_hbm.at[pl.ds(r0, ns)], idxd)
            pltpu.sync_copy(src_hbm.at[pl.ds(r0, ns)], idxs)
            cps = []
            for j in range(ns):
                sl = pl.ds(j * sub, sub)
                cps.append(pltpu.async_copy(x_hbm.at[idxd.at[j]], bi.at[sl], sem))
                cps.append(pltpu.async_copy(x_hbm.at[idxs.at[j]], bj.at[sl], sem))
            for cp in cps:
                cp.wait()
            e0 = r0 * sub
            pltpu.sync_copy(bi, out_hbm.at[pl.ds(e0, ch), pl.ds(0, w)])
            pltpu.sync_copy(bj, out_hbm.at[pl.ds(e0, ch), pl.ds(w, w)])
            return 0

        lax.fori_loop(0, rpw // ns, chunk, 0)

    return k(x, dst2, src2)


def _sc_scatter_chsplit(msg, dst2, nnodes):
    nnodes = int(nnodes)
    d = msg.shape[1]
    dch = d // 2
    ngrp = -(-dch * nnodes // 900_000)
    while dch % ngrp:
        ngrp += 1
    dcg = dch // ngrp
    rows, sub = dst2.shape
    rps = rows // 16
    ns = _nsub(rps)
    ch = ns * sub
    npr = nnodes // 16
    nz = 5 if npr % 5 == 0 and npr > 1024 else 1
    zr = npr // nz
    mesh = plsc.VectorSubcoreMesh(**_MESH)

    @functools.partial(
        pl.kernel, mesh=mesh,
        out_type=jax.ShapeDtypeStruct((nnodes, d), jnp.float32),
        compiler_params=_SC_PARAMS,
        scratch_types=[
            pltpu.VMEM((ns, sub), jnp.int32),
            pltpu.VMEM((ch, dcg), jnp.float32),
            pltpu.VMEM((zr, dcg), jnp.float32),
            pltpu.VMEM_SHARED((nnodes, dcg), jnp.float32),
            pltpu.SemaphoreType.DMA,
        ],
    )
    def k(msg_hbm, dst_hbm, out_hbm, idxd, vals, zb, acc, sem):
        cid = lax.axis_index("c")
        sid = lax.axis_index("s")

        for g in range(ngrp):
            c0 = cid * dch + g * dcg
            zb[...] = jnp.zeros((zr, dcg), jnp.float32)

            def zero(c, _):
                pltpu.sync_copy(zb, acc.at[pl.ds(sid * npr + c * zr, zr)])
                return 0

            lax.fori_loop(0, nz, zero, 0)
            plsc.subcore_barrier()

            def chunk(c, _):
                r0 = sid * rps + c * ns
                pltpu.sync_copy(dst_hbm.at[pl.ds(r0, ns)], idxd)
                pltpu.sync_copy(
                    msg_hbm.at[pl.ds(r0 * sub, ch), pl.ds(c0, dcg)], vals)
                for j in range(ns):
                    pltpu.sync_copy(vals.at[pl.ds(j * sub, sub)],
                                    acc.at[idxd.at[j]], add=True)
                return 0

            lax.fori_loop(0, rps // ns, chunk, 0)
            plsc.subcore_barrier()

            def drain(c, _):
                r0 = sid * npr + c * zr
                pltpu.sync_copy(acc.at[pl.ds(r0, zr)], zb)
                pltpu.sync_copy(zb, out_hbm.at[pl.ds(r0, zr), pl.ds(c0, dcg)])
                return 0

            lax.fori_loop(0, nz, drain, 0)
            plsc.subcore_barrier()

    return k(msg, dst2)


def _sc_scatter_edgesplit(msg, dst2, nnodes):
    nnodes = int(nnodes)
    d = msg.shape[1]
    assert d == 16
    rows, sub = dst2.shape
    rps = rows // NW
    ns = _nsub(rps)
    ch = ns * sub
    npr = nnodes // 16
    nz = 5 if npr % 5 == 0 and npr > 1024 else 1
    zr = npr // nz
    mesh = plsc.VectorSubcoreMesh(**_MESH)

    @functools.partial(
        pl.kernel, mesh=mesh,
        out_type=jax.ShapeDtypeStruct((2, nnodes, d), jnp.float32),
        compiler_params=_SC_PARAMS,
        scratch_types=[
            pltpu.VMEM((ns, sub), jnp.int32),
            pltpu.VMEM((ch, d), jnp.float32),
            pltpu.VMEM((zr, d), jnp.float32),
            pltpu.VMEM_SHARED((nnodes, d), jnp.float32),
            pltpu.SemaphoreType.DMA,
        ],
    )
    def k(msg_hbm, dst_hbm, out_hbm, idxd, vals, zb, acc, sem):
        cid = lax.axis_index("c")
        sid = lax.axis_index("s")
        zb[...] = jnp.zeros((zr, d), jnp.float32)

        def zero(c, _):
            pltpu.sync_copy(zb, acc.at[pl.ds(sid * npr + c * zr, zr)])
            return 0

        lax.fori_loop(0, nz, zero, 0)
        plsc.subcore_barrier()

        def chunk(c, _):
            r0 = (cid * 16 + sid) * rps + c * ns
            pltpu.sync_copy(dst_hbm.at[pl.ds(r0, ns)], idxd)
            pltpu.sync_copy(msg_hbm.at[pl.ds(r0 * sub, ch)], vals)
            for j in range(ns):
                pltpu.sync_copy(vals.at[pl.ds(j * sub, sub)],
                                acc.at[idxd.at[j]], add=True)
            return 0

        lax.fori_loop(0, rps // ns, chunk, 0)
        plsc.subcore_barrier()

        def drain(c, _):
            r0 = sid * npr + c * zr
            pltpu.sync_copy(acc.at[pl.ds(r0, zr)], zb)
            pltpu.sync_copy(zb, out_hbm.at[cid, pl.ds(r0, zr)])
            return 0

        lax.fori_loop(0, nz, drain, 0)

    return k(msg, dst2)


def _sc_count(dst2, nnodes):
    nnodes = int(nnodes)
    rows, sub = dst2.shape
    rps = rows // NW
    ns = _nsub(rps)
    ch = ns * sub
    npr = nnodes // 16
    nz = 5 if npr % 5 == 0 and npr > 1024 else 1
    zr = npr // nz
    mesh = plsc.VectorSubcoreMesh(**_MESH)

    @functools.partial(
        pl.kernel, mesh=mesh,
        out_type=jax.ShapeDtypeStruct((2, nnodes, 16), jnp.float32),
        compiler_params=_SC_PARAMS,
        scratch_types=[
            pltpu.VMEM((ns, sub), jnp.int32),
            pltpu.VMEM((ch, 16), jnp.float32),
            pltpu.VMEM((zr, 16), jnp.float32),
            pltpu.VMEM_SHARED((nnodes, 16), jnp.float32),
            pltpu.SemaphoreType.DMA,
        ],
    )
    def k(dst_hbm, out_hbm, idxd, ones, zb, acc, sem):
        cid = lax.axis_index("c")
        sid = lax.axis_index("s")
        zb[...] = jnp.zeros((zr, 16), jnp.float32)
        ones[...] = jnp.ones((ch, 16), jnp.float32)

        def zero(c, _):
            pltpu.sync_copy(zb, acc.at[pl.ds(sid * npr + c * zr, zr)])
            return 0

        lax.fori_loop(0, nz, zero, 0)
        plsc.subcore_barrier()

        def chunk(c, _):
            r0 = (cid * 16 + sid) * rps + c * ns
            pltpu.sync_copy(dst_hbm.at[pl.ds(r0, ns)], idxd)
            for j in range(ns):
                pltpu.sync_copy(ones.at[pl.ds(j * sub, sub)],
                                acc.at[idxd.at[j]], add=True)
            return 0

        lax.fori_loop(0, rps // ns, chunk, 0)
        plsc.subcore_barrier()

        def drain(c, _):
            r0 = sid * npr + c * zr
            pltpu.sync_copy(acc.at[pl.ds(r0, zr)], zb)
            pltpu.sync_copy(zb, out_hbm.at[cid, pl.ds(r0, zr)])
            return 0

        lax.fori_loop(0, nz, drain, 0)

    return k(dst2)


def _fold_stats(pref, gamma, beta, nedges):
    part = pref[...]
    m = jnp.sum(part[:, 0, :], axis=0) / nedges
    v = jnp.sum(part[:, 1, :], axis=0) / nedges - m * m
    scale = gamma * jax.lax.rsqrt(v + EPS)
    shift = beta - m * scale
    return scale, shift


def _tc_pass(xc, partials, weights, bns, chain, out_dim, final, nedges):
    e, din = xc.shape
    be = 8000 if e % 8000 == 0 else e // 8
    nb = e // be
    nprev = len(partials)

    def body(*refs):
        xref = refs[0]
        prefs = refs[1:1 + nprev]
        bnrefs = refs[1 + nprev:1 + 3 * nprev]
        wrefs = refs[1 + 3 * nprev:-2]
        outref = refs[-2]
        st = refs[-1]

        @pl.when(pl.program_id(0) == 0)
        def _():
            for i, pref in enumerate(prefs):
                di = pref.shape[2]
                scale, shift = _fold_stats(
                    pref, bnrefs[2 * i][...], bnrefs[2 * i + 1][...], nedges)
                st[i, 0, :di] = scale
                st[i, 1, :di] = shift

        affines = []
        for i, pref in enumerate(prefs):
            di = pref.shape[2]
            affines.append((st[i, 0, :di], st[i, 1, :di]))
        act = chain(xref[...], [w[...] for w in wrefs], affines)
        if final:
            outref[...] = act
        else:
            outref[0, 0, :] = jnp.sum(act, axis=0)
            outref[0, 1, :] = jnp.sum(act * act, axis=0)

    full = lambda a: pl.BlockSpec(a.shape, lambda b: (0,) * a.ndim)
    in_specs = [pl.BlockSpec((be, din), lambda b: (b, 0))]
    in_specs += [full(p) for p in partials]
    bn_flat = [g for pair in bns for g in pair]
    in_specs += [full(g) for g in bn_flat]
    in_specs += [full(w) for w in weights]
    if final:
        out_shape = jax.ShapeDtypeStruct((e, out_dim), jnp.float32)
        out_spec = pl.BlockSpec((be, out_dim), lambda b: (b, 0))
    else:
        out_shape = jax.ShapeDtypeStruct((nb, 2, out_dim), jnp.float32)
        out_spec = pl.BlockSpec((1, 2, out_dim), lambda b: (b, 0, 0))

    return pl.pallas_call(
        body,
        grid=(nb,),
        in_specs=in_specs,
        out_specs=out_spec,
        out_shape=out_shape,
        scratch_shapes=[pltpu.VMEM((max(nprev, 1), 2, 64), jnp.float32)],
    )(xc, *partials, *bn_flat, *weights)


def _nblk(n):
    return 5000 if n % 5000 == 0 else n


def _tc_bn0(x, gamma, beta):
    n, f = x.shape
    xr = x.reshape(n // 8, 8 * f)

    def body(xref, gref, bref, outref):
        xv = xref[...]
        m8 = jnp.mean(xv, axis=0)
        q8 = jnp.mean(xv * xv, axis=0)
        m = m8[0:f]
        q = q8[0:f]
        for kk in range(1, 8):
            m = m + m8[kk * f:(kk + 1) * f]
            q = q + q8[kk * f:(kk + 1) * f]
        m = m / 8.0
        v = q / 8.0 - m * m
        scale = jax.lax.rsqrt(v + EPS) * gref[...]
        shift = bref[...] - m * scale
        outref[...] = (xv * jnp.concatenate([scale] * 8)
                       + jnp.concatenate([shift] * 8))

    out = pl.pallas_call(
        body,
        out_shape=jax.ShapeDtypeStruct(xr.shape, jnp.float32),
    )(xr, gamma, beta)
    return out.reshape(n, f)


def _tc_inv(cnts):
    _, n, w = cnts.shape
    bn = _nblk(n)

    def body(cref, outref):
        outref[...] = 1.0 / jnp.maximum(
            cref[0, :, 0:1] + cref[1, :, 0:1], 1.0)

    return pl.pallas_call(
        body,
        grid=(n // bn,),
        in_specs=[pl.BlockSpec((2, bn, w), lambda b: (0, b, 0))],
        out_specs=pl.BlockSpec((bn, 1), lambda b: (b, 0)),
        out_shape=jax.ShapeDtypeStruct((n, 1), jnp.float32),
    )(cnts)


def _tc_pass_first(xc, weights, chain0, dmat, dstat, nedges):
    e, din = xc.shape
    be = 8000 if e % 8000 == 0 else e // 8
    nb = e // be

    def body(xref, *rest):
        wrefs = rest[:-2]
        matref = rest[-2]
        outref = rest[-1]
        mat, act = chain0(xref[...], [w[...] for w in wrefs])
        matref[...] = mat
        outref[0, 0, :] = jnp.sum(act, axis=0)
        outref[0, 1, :] = jnp.sum(act * act, axis=0)

    full = lambda a: pl.BlockSpec(a.shape, lambda b: (0,) * a.ndim)
    return pl.pallas_call(
        body,
        grid=(nb,),
        in_specs=[pl.BlockSpec((be, din), lambda b: (b, 0))]
        + [full(w) for w in weights],
        out_specs=(pl.BlockSpec((be, dmat), lambda b: (b, 0)),
                   pl.BlockSpec((1, 2, dstat), lambda b: (b, 0, 0))),
        out_shape=(jax.ShapeDtypeStruct((e, dmat), jnp.float32),
                   jax.ShapeDtypeStruct((nb, 2, dstat), jnp.float32)),
    )(xc, *weights)


def _tc_mean(acc2, inv, dout=None):
    _, n, dn = acc2.shape
    dout = dn if dout is None else dout
    bn = _nblk(n)

    def body(aref, iref, outref):
        outref[...] = (aref[0, :, 0:dout] + aref[1, :, 0:dout]) * iref[...]

    return pl.pallas_call(
        body,
        grid=(n // bn,),
        in_specs=[pl.BlockSpec((2, bn, dn), lambda b: (0, b, 0)),
                  pl.BlockSpec((bn, 1), lambda b: (b, 0))],
        out_specs=pl.BlockSpec((bn, dout), lambda b: (b, 0)),
        out_shape=jax.ShapeDtypeStruct((n, dout), jnp.float32),
    )(acc2, inv)


def _tc_mean_chsplit(acc, inv):
    n, dn = acc.shape
    bn = _nblk(n)

    def body(aref, iref, outref):
        outref[...] = aref[...] * iref[...]

    return pl.pallas_call(
        body,
        grid=(n // bn,),
        in_specs=[pl.BlockSpec((bn, dn), lambda b: (b, 0)),
                  pl.BlockSpec((bn, 1), lambda b: (b, 0))],
        out_specs=pl.BlockSpec((bn, dn), lambda b: (b, 0)),
        out_shape=jax.ShapeDtypeStruct((n, dn), jnp.float32),
    )(acc, inv)


def _first_chain(f, w, kind):
    def chain(xb, ws):
        xi = xb[:, 0:f]
        xj = xb[:, w:w + f]
        t = jnp.concatenate([xi, xj - xi], axis=1)
        h0 = jnp.dot(t, ws[0], preferred_element_type=jnp.float32) + ws[1]
        a0 = jnp.maximum(h0, 0.0)
        if kind == "t":
            return t, a0
        if kind == "h0":
            return h0, a0
        h1 = jnp.dot(a0, ws[2], preferred_element_type=jnp.float32) + ws[3]
        return t, h1

    return chain


def _enc_chain(stage):
    def chain(xb, ws, affines):
        w0, b0, w1, b1, w2, b2 = ws
        a = jnp.maximum(jnp.dot(xb, w0, preferred_element_type=jnp.float32)
                        + b0, 0.0)
        a = a * affines[0][0] + affines[0][1]
        a = jnp.maximum(jnp.dot(a, w1, preferred_element_type=jnp.float32)
                        + b1, 0.0)
        if stage == 1:
            return a
        a = a * affines[1][0] + affines[1][1]
        a = jnp.maximum(jnp.dot(a, w2, preferred_element_type=jnp.float32)
                        + b2, 0.0)
        if stage == 2:
            return a
        return a * affines[2][0] + affines[2][1]

    return chain


def _enc_chain_pre(stage):
    def chain(xb, ws, affines):
        w1, b1, w2, b2 = ws
        a = jnp.maximum(xb, 0.0)
        a = a * affines[0][0] + affines[0][1]
        a = jnp.maximum(jnp.dot(a, w1, preferred_element_type=jnp.float32)
                        + b1, 0.0)
        if stage == 1:
            return a
        a = a * affines[1][0] + affines[1][1]
        a = jnp.maximum(jnp.dot(a, w2, preferred_element_type=jnp.float32)
                        + b2, 0.0)
        if stage == 2:
            return a
        return a * affines[2][0] + affines[2][1]

    return chain


def _dec1_chain(stage):
    def chain(xb, ws, affines):
        w0, b0, w1, b1, w2, b2 = ws
        a = jnp.maximum(jnp.dot(xb, w0, preferred_element_type=jnp.float32)
                        + b0, 0.0)
        h1 = jnp.dot(a, w1, preferred_element_type=jnp.float32) + b1
        a = jnp.maximum(h1 * affines[0][0] + affines[0][1], 0.0)
        a = jnp.maximum(jnp.dot(a, w2, preferred_element_type=jnp.float32)
                        + b2, 0.0)
        if stage == 1:
            return a
        return a * affines[1][0] + affines[1][1]

    return chain


def _dec2_chain(stage):
    def chain(xb, ws, affines):
        w1, b1, w2, b2 = ws
        a = jnp.maximum(xb, 0.0)
        a = a * affines[0][0] + affines[0][1]
        a = jnp.maximum(jnp.dot(a, w1, preferred_element_type=jnp.float32)
                        + b1, 0.0)
        if stage == 1:
            return a
        a = a * affines[1][0] + affines[1][1]
        return jnp.dot(a, w2, preferred_element_type=jnp.float32) + b2

    return chain


def _pad16_chain(fn):
    def chain(xb, ws, affines):
        r = fn(xb, ws, affines)
        z = jnp.zeros((r.shape[0], 16 - r.shape[1]), r.dtype)
        return jnp.concatenate([r, z], axis=1)

    return chain


def kernel(x, edge_index, params):
    p = params
    n, f = x.shape
    e = edge_index.shape[1]
    dst2 = edge_index[1].reshape(e // SUB, SUB)
    src2 = edge_index[0].reshape(e // SUB, SUB)

    cnts = _sc_count(dst2, n)
    inv = _tc_inv(cnts)

    h = _tc_bn0(x, p["bn0_g"], p["bn0_b"])
    h = jnp.pad(h, ((0, 0), (0, 16 - f)))

    ws = [p["e1_w0"], p["e1_b0"], p["e1_w1"], p["e1_b1"],
          p["e1_w2"], p["e1_b2"]]
    bns = [(p["e1_g0"], p["e1_bb0"]), (p["e1_g1"], p["e1_bb1"]),
           (p["e1_g2"], p["e1_bb2"])]
    xc = _sc_gather_concat(h, dst2, src2)
    t1, p0 = _tc_pass_first(xc, ws[:2], _first_chain(f, 16, "t"), 2 * f, 64, e)
    p1 = _tc_pass(t1, [p0], ws, bns[:1], _enc_chain(1), 64, False, e)
    p2 = _tc_pass(t1, [p0, p1], ws, bns[:2], _enc_chain(2), 32, False, e)
    msg = _tc_pass(t1, [p0, p1, p2], ws, bns, _enc_chain(3), 32, True, e)
    acc = _sc_scatter_chsplit(msg, dst2, n)
    x1 = _tc_mean_chsplit(acc, inv)

    ws = [p["e2_w1"], p["e2_b1"], p["e2_w2"], p["e2_b2"]]
    bns = [(p["e2_g0"], p["e2_bb0"]), (p["e2_g1"], p["e2_bb1"]),
           (p["e2_g2"], p["e2_bb2"])]
    xc = _sc_gather_concat(x1, dst2, src2)
    h0, p0 = _tc_pass_first(xc, [p["e2_w0"], p["e2_b0"]],
                            _first_chain(32, 32, "h0"), 32, 32, e)
    p1 = _tc_pass(h0, [p0], ws, bns[:1], _enc_chain_pre(1), 32, False, e)
    p2 = _tc_pass(h0, [p0, p1], ws, bns[:2], _enc_chain_pre(2), 2, False, e)
    msg = _tc_pass(h0, [p0, p1, p2], ws, bns,
                   _pad16_chain(_enc_chain_pre(3)), 16, True, e)
    acc2 = _sc_scatter_edgesplit(msg, dst2, n)
    x2 = _tc_mean(acc2, inv)

    ws = [p["d1_w0"], p["d1_b0"], p["d1_w1"], p["d1_b1"],
          p["d1_w2"], p["d1_b2"]]
    bns = [(p["d1_g0"], p["d1_bb0"]), (p["d1_g1"], p["d1_bb1"])]
    xc = _sc_gather_concat(x2, dst2, src2)
    t3, p0 = _tc_pass_first(xc, ws[:4], _first_chain(2, 16, "dec1"), 4, 32, e)
    p1 = _tc_pass(t3, [p0], ws, bns[:1], _dec1_chain(1), 64, False, e)
    msg = _tc_pass(t3, [p0, p1], ws, bns, _dec1_chain(2), 64, True, e)
    acc = _sc_scatter_chsplit(msg, dst2, n)
    x3 = _tc_mean_chsplit(acc, inv)

    ws = [p["d2_w1"], p["d2_b1"], p["d2_w2"], p["d2_b2"]]
    bns = [(p["d2_g0"], p["d2_bb0"]), (p["d2_g1"], p["d2_bb1"])]
    xc = _sc_gather_concat(x3, dst2, src2)
    h0, p0 = _tc_pass_first(xc, [p["d2_w0"], p["d2_b0"]],
                            _first_chain(64, 64, "h0"), 64, 64, e)
    p1 = _tc_pass(h0, [p0], ws, bns[:1], _dec2_chain(1), 64, False, e)
    msg = _tc_pass(h0, [p0, p1], ws, bns,
                   _pad16_chain(_dec2_chain(2)), 16, True, e)
    acc2 = _sc_scatter_edgesplit(msg, dst2, n)
    return _tc_mean(acc2, inv, dout=4)
